# baseline (speedup 1.0000x reference)
import functools

import jax
import jax.numpy as jnp
from jax import lax
from jax.experimental import pallas as pl

NEG_POS_RATIO = 3
NUM_CLASSES = 20


def _phase1(loc_p_ref, loc_t_ref, cls_p_ref, cls_t_ref, ce_ref, stats_ref, *, C):
    nc = pl.program_id(1)

    t = cls_t_ref[0]
    pos = t > 0
    neg = t == 0

    d = jnp.transpose(loc_p_ref[0] - loc_t_ref[0])
    ad = jnp.abs(d)
    sl1 = jnp.where(ad < 1.0, 0.5 * d * d, ad - 0.5)
    sl = jnp.sum(sl1, axis=0, keepdims=True)
    loc_part = jnp.sum(jnp.where(pos, sl, 0.0))

    x = jnp.transpose(cls_p_ref[0])
    m = jnp.max(x, axis=0, keepdims=True)
    lse = m + jnp.log(jnp.sum(jnp.exp(x - m), axis=0, keepdims=True))
    ci = lax.broadcasted_iota(jnp.int32, x.shape, 0)
    xt = jnp.sum(jnp.where(ci == t, x, 0.0), axis=0, keepdims=True)
    ce = lse - xt

    posce_part = jnp.sum(jnp.where(pos, ce, 0.0))
    ce_ref[0] = jnp.where(neg, ce, -jnp.inf)

    @pl.when(nc == 0)
    def _():
        stats_ref[...] = jnp.zeros_like(stats_ref)

    posf = jnp.sum(pos.astype(jnp.float32))
    negf = jnp.sum(neg.astype(jnp.float32))
    stats_ref[...] += jnp.stack([posf, negf, loc_part, posce_part]).reshape(1, 1, 4)


def _phase2(ce_ref, stats_ref, out_ref):
    v = ce_ref[...]
    vi = lax.bitcast_convert_type(v, jnp.int32)
    stats = stats_ref[...]
    num_pos = stats[:, 0:1]
    num_neg = stats[:, 1:2]
    k = jnp.minimum(NEG_POS_RATIO * num_pos, num_neg)

    B = v.shape[0]
    lo = jnp.zeros((B, 1), jnp.int32)
    hi = jnp.full((B, 1), 2**31 - 2, jnp.int32)

    def body(_, carry):
        lo, hi = carry
        mid = lo + (hi - lo + 1) // 2
        cnt = jnp.sum((vi >= mid).astype(jnp.float32), axis=1, keepdims=True)
        ok = cnt >= k
        return jnp.where(ok, mid, lo), jnp.where(ok, hi, mid - 1)

    lo, hi = lax.fori_loop(0, 31, body, (lo, hi))
    tstar = lo
    gt = vi > tstar
    c_gt = jnp.sum(gt.astype(jnp.float32), axis=1, keepdims=True)
    sum_gt = jnp.sum(jnp.where(gt, v, 0.0), axis=1, keepdims=True)
    tf = lax.bitcast_convert_type(tstar, jnp.float32)
    topk = sum_gt + (k - c_gt) * tf
    topk = jnp.where(k > 0.5, topk, 0.0)

    loc_loss = jnp.sum(stats[:, 2])
    posce = jnp.sum(stats[:, 3])
    n = jnp.maximum(jnp.sum(num_pos), 1.0)
    out_ref[...] = jnp.reshape((loc_loss + posce + jnp.sum(topk)) / n, (1, 1))


def kernel(loc_preds, cls_preds, loc_targets, cls_targets):
    B, P = loc_preds.shape[0], loc_preds.shape[1]
    C = 8192
    NC = (P + C - 1) // C
    Ppad = NC * C

    cls_t = jnp.pad(cls_targets.astype(jnp.int32), ((0, 0), (0, Ppad - P)),
                    constant_values=-1).reshape(B * NC, 1, C)

    ce_neg, stats = pl.pallas_call(
        functools.partial(_phase1, C=C),
        grid=(B, NC),
        in_specs=[
            pl.BlockSpec((1, C, 4), lambda b, nc: (b, nc, 0)),
            pl.BlockSpec((1, C, 4), lambda b, nc: (b, nc, 0)),
            pl.BlockSpec((1, C, NUM_CLASSES + 1), lambda b, nc: (b, nc, 0)),
            pl.BlockSpec((1, 1, C), lambda b, nc: (b * NC + nc, 0, 0)),
        ],
        out_specs=[
            pl.BlockSpec((1, 1, C), lambda b, nc: (b * NC + nc, 0, 0)),
            pl.BlockSpec((1, 1, 4), lambda b, nc: (b, 0, 0)),
        ],
        out_shape=[
            jax.ShapeDtypeStruct((B * NC, 1, C), jnp.float32),
            jax.ShapeDtypeStruct((B, 1, 4), jnp.float32),
        ],
    )(loc_preds, loc_targets, cls_preds, cls_t)

    out = pl.pallas_call(
        _phase2,
        out_shape=jax.ShapeDtypeStruct((1, 1), jnp.float32),
    )(ce_neg.reshape(B, Ppad), stats.reshape(B, 4))

    return out.reshape(())


# speedup vs baseline: 3.4372x; 3.4372x over previous
import functools

import jax
import jax.numpy as jnp
from jax import lax
from jax.experimental import pallas as pl

NEG_POS_RATIO = 3
NUM_CLASSES = 20


def _phase1(loc_p_ref, loc_t_ref, cls_p_ref, cls_t_ref, t2_ref,
            ce_ref, stats_ref, *, C, P4):
    t = cls_t_ref[0]
    pos = t > 0
    neg = t == 0

    li = lax.broadcasted_iota(jnp.int32, (1, 4 * C), 1)
    d = jnp.where(li < P4, loc_p_ref[0] - loc_t_ref[0], 0.0)
    ad = jnp.abs(d)
    sl1 = jnp.where(ad < 1.0, 0.5 * d * d, ad - 0.5)
    rows = sl1.reshape(-1, 128)
    ri = lax.broadcasted_iota(jnp.int32, (128, 32), 0)
    cj = lax.broadcasted_iota(jnp.int32, (128, 32), 1)
    S = (ri // 4 == cj).astype(jnp.float32)
    s2 = lax.dot_general(rows, S, (((1,), (0,)), ((), ())),
                         precision=lax.Precision.HIGHEST)
    posM = t2_ref[0] > 0
    loc_part = jnp.sum(jnp.where(posM, s2, 0.0))

    x = cls_p_ref[0]
    m = jnp.max(x, axis=0, keepdims=True)
    lse = m + jnp.log(jnp.sum(jnp.exp(x - m), axis=0, keepdims=True))
    ci = lax.broadcasted_iota(jnp.int32, x.shape, 0)
    xt = jnp.sum(jnp.where(ci == t, x, 0.0), axis=0, keepdims=True)
    ce = lse - xt

    posce_part = jnp.sum(jnp.where(pos, ce, 0.0))
    ce_ref[0] = jnp.where(neg, ce, -jnp.inf)

    posf = jnp.sum(pos.astype(jnp.float32))
    negf = jnp.sum(neg.astype(jnp.float32))
    stats_ref[...] = jnp.stack([posf, negf, loc_part, posce_part]).reshape(1, 1, 4)


def _phase2(ce_ref, stats_ref, out_ref):
    v = ce_ref[...]
    vi = lax.bitcast_convert_type(v, jnp.int32)
    stats = stats_ref[...]
    num_pos = stats[:, 0:1]
    num_neg = stats[:, 1:2]
    k = jnp.minimum(NEG_POS_RATIO * num_pos, num_neg)

    B = v.shape[0]
    lo = jnp.zeros((B, 1), jnp.int32)
    hi = jnp.full((B, 1), 2**31 - 2, jnp.int32)

    def body(_, carry):
        lo, hi = carry
        mid = lo + (hi - lo + 1) // 2
        cnt = jnp.sum((vi >= mid).astype(jnp.float32), axis=1, keepdims=True)
        ok = cnt >= k
        return jnp.where(ok, mid, lo), jnp.where(ok, hi, mid - 1)

    lo, hi = lax.fori_loop(0, 31, body, (lo, hi))
    tstar = lo
    gt = vi > tstar
    c_gt = jnp.sum(gt.astype(jnp.float32), axis=1, keepdims=True)
    sum_gt = jnp.sum(jnp.where(gt, v, 0.0), axis=1, keepdims=True)
    tf = lax.bitcast_convert_type(tstar, jnp.float32)
    topk = sum_gt + (k - c_gt) * tf
    topk = jnp.where(k > 0.5, topk, 0.0)

    loc_loss = jnp.sum(stats[:, 2])
    posce = jnp.sum(stats[:, 3])
    n = jnp.maximum(jnp.sum(num_pos), 1.0)
    out_ref[...] = jnp.reshape((loc_loss + posce + jnp.sum(topk)) / n, (1, 1))


def kernel(loc_preds, cls_preds, loc_targets, cls_targets):
    B, P = loc_preds.shape[0], loc_preds.shape[1]
    C = 24576
    R = 4 * C // 128

    ct_pad = jnp.pad(cls_targets.astype(jnp.int32), ((0, 0), (0, C - P)),
                     constant_values=-1)
    cls_t = ct_pad.reshape(B, 1, C)
    t2 = ct_pad.reshape(B, R, 32)

    loc_p_f = loc_preds.reshape(B, 1, 4 * P)
    loc_t_f = loc_targets.reshape(B, 1, 4 * P)
    cls_p_t = jnp.transpose(cls_preds, (0, 2, 1))

    ce_neg, stats = pl.pallas_call(
        functools.partial(_phase1, C=C, P4=4 * P),
        grid=(B,),
        in_specs=[
            pl.BlockSpec((1, 1, 4 * C), lambda b: (b, 0, 0)),
            pl.BlockSpec((1, 1, 4 * C), lambda b: (b, 0, 0)),
            pl.BlockSpec((1, NUM_CLASSES + 1, C), lambda b: (b, 0, 0)),
            pl.BlockSpec((1, 1, C), lambda b: (b, 0, 0)),
            pl.BlockSpec((1, R, 32), lambda b: (b, 0, 0)),
        ],
        out_specs=[
            pl.BlockSpec((1, 1, C), lambda b: (b, 0, 0)),
            pl.BlockSpec((1, 1, 4), lambda b: (b, 0, 0)),
        ],
        out_shape=[
            jax.ShapeDtypeStruct((B, 1, C), jnp.float32),
            jax.ShapeDtypeStruct((B, 1, 4), jnp.float32),
        ],
    )(loc_p_f, loc_t_f, cls_p_t, cls_t, t2)

    out = pl.pallas_call(
        _phase2,
        out_shape=jax.ShapeDtypeStruct((1, 1), jnp.float32),
    )(ce_neg.reshape(B, C), stats.reshape(B, 4))

    return out.reshape(())


# speedup vs baseline: 3.7410x; 1.0884x over previous
import functools

import jax
import jax.numpy as jnp
from jax import lax
from jax.experimental import pallas as pl

NEG_POS_RATIO = 3
NUM_CLASSES = 20


def _phase1(loc_p_ref, loc_t_ref, cls_p_ref, cls_t_ref, ce_ref, stats_ref, *, C):
    nc = pl.program_id(1)

    t = cls_t_ref[0]
    pos = t > 0
    neg = t == 0

    d = loc_p_ref[0] - loc_t_ref[0]
    ad = jnp.abs(d)
    sl1 = jnp.where(ad < 1.0, 0.5 * d * d, ad - 0.5)
    sl = jnp.sum(sl1, axis=0, keepdims=True)
    loc_part = jnp.sum(jnp.where(pos, sl, 0.0))

    x = cls_p_ref[0]
    m = jnp.max(x, axis=0, keepdims=True)
    lse = m + jnp.log(jnp.sum(jnp.exp(x - m), axis=0, keepdims=True))
    ci = lax.broadcasted_iota(jnp.int32, x.shape, 0)
    xt = jnp.sum(jnp.where(ci == t, x, 0.0), axis=0, keepdims=True)
    ce = lse - xt

    posce_part = jnp.sum(jnp.where(pos, ce, 0.0))
    ce_ref[0] = jnp.where(neg, ce, -jnp.inf)

    @pl.when(nc == 0)
    def _():
        stats_ref[...] = jnp.zeros_like(stats_ref)

    posf = jnp.sum(pos.astype(jnp.float32))
    negf = jnp.sum(neg.astype(jnp.float32))
    stats_ref[...] += jnp.stack([posf, negf, loc_part, posce_part]).reshape(1, 1, 4)


def _phase2(ce_ref, stats_ref, out_ref):
    v = ce_ref[...]
    vi = lax.bitcast_convert_type(v, jnp.int32)
    stats = stats_ref[...]
    num_pos = stats[:, 0:1]
    num_neg = stats[:, 1:2]
    k = jnp.minimum(NEG_POS_RATIO * num_pos, num_neg)

    B = v.shape[0]
    lo = jnp.zeros((B, 1), jnp.int32)
    hi = jnp.full((B, 1), 2**31 - 2, jnp.int32)

    def body(_, carry):
        lo, hi = carry
        mid = lo + (hi - lo + 1) // 2
        cnt = jnp.sum((vi >= mid).astype(jnp.float32), axis=1, keepdims=True)
        ok = cnt >= k
        return jnp.where(ok, mid, lo), jnp.where(ok, hi, mid - 1)

    lo, hi = lax.fori_loop(0, 31, body, (lo, hi))
    tstar = lo
    gt = vi > tstar
    c_gt = jnp.sum(gt.astype(jnp.float32), axis=1, keepdims=True)
    sum_gt = jnp.sum(jnp.where(gt, v, 0.0), axis=1, keepdims=True)
    tf = lax.bitcast_convert_type(tstar, jnp.float32)
    topk = sum_gt + (k - c_gt) * tf
    topk = jnp.where(k > 0.5, topk, 0.0)

    loc_loss = jnp.sum(stats[:, 2])
    posce = jnp.sum(stats[:, 3])
    n = jnp.maximum(jnp.sum(num_pos), 1.0)
    out_ref[...] = jnp.reshape((loc_loss + posce + jnp.sum(topk)) / n, (1, 1))


def kernel(loc_preds, cls_preds, loc_targets, cls_targets):
    B, P = loc_preds.shape[0], loc_preds.shape[1]
    C = 4096
    NC = (P + C - 1) // C
    Ppad = NC * C

    cls_t = jnp.pad(cls_targets.astype(jnp.int32), ((0, 0), (0, Ppad - P)),
                    constant_values=-1).reshape(B * NC, 1, C)

    loc_p_t = jnp.transpose(loc_preds, (0, 2, 1))
    loc_t_t = jnp.transpose(loc_targets, (0, 2, 1))
    cls_p_t = jnp.transpose(cls_preds, (0, 2, 1))

    ce_neg, stats = pl.pallas_call(
        functools.partial(_phase1, C=C),
        grid=(B, NC),
        in_specs=[
            pl.BlockSpec((1, 4, C), lambda b, nc: (b, 0, nc)),
            pl.BlockSpec((1, 4, C), lambda b, nc: (b, 0, nc)),
            pl.BlockSpec((1, NUM_CLASSES + 1, C), lambda b, nc: (b, 0, nc)),
            pl.BlockSpec((1, 1, C), lambda b, nc: (b * NC + nc, 0, 0)),
        ],
        out_specs=[
            pl.BlockSpec((1, 1, C), lambda b, nc: (b * NC + nc, 0, 0)),
            pl.BlockSpec((1, 1, 4), lambda b, nc: (b, 0, 0)),
        ],
        out_shape=[
            jax.ShapeDtypeStruct((B * NC, 1, C), jnp.float32),
            jax.ShapeDtypeStruct((B, 1, 4), jnp.float32),
        ],
    )(loc_p_t, loc_t_t, cls_p_t, cls_t)

    out = pl.pallas_call(
        _phase2,
        out_shape=jax.ShapeDtypeStruct((1, 1), jnp.float32),
    )(ce_neg.reshape(B, Ppad), stats.reshape(B, 4))

    return out.reshape(())


# speedup vs baseline: 4.9941x; 1.3350x over previous
import functools

import jax
import jax.numpy as jnp
from jax import lax
from jax.experimental import pallas as pl

NEG_POS_RATIO = 3
NUM_CLASSES = 20


def _phase1(loc_p_ref, loc_t_ref, cls_p_ref, cls_t_ref, ce_ref, stats_ref, *, C):
    nc = pl.program_id(1)

    t = cls_t_ref[0]
    pos = t > 0
    neg = t == 0

    d = loc_p_ref[0] - loc_t_ref[0]
    ad = jnp.abs(d)
    sl1 = jnp.where(ad < 1.0, 0.5 * d * d, ad - 0.5)
    sl = jnp.sum(sl1, axis=0, keepdims=True)
    loc_part = jnp.sum(jnp.where(pos, sl, 0.0))

    x = cls_p_ref[0]
    m = jnp.max(x, axis=0, keepdims=True)
    lse = m + jnp.log(jnp.sum(jnp.exp(x - m), axis=0, keepdims=True))
    ci = lax.broadcasted_iota(jnp.int32, x.shape, 0)
    xt = jnp.sum(jnp.where(ci == t, x, 0.0), axis=0, keepdims=True)
    ce = lse - xt

    posce_part = jnp.sum(jnp.where(pos, ce, 0.0))
    ce_ref[0] = jnp.where(neg, ce, -jnp.inf)

    @pl.when(nc == 0)
    def _():
        stats_ref[...] = jnp.zeros_like(stats_ref)

    posf = jnp.sum(pos.astype(jnp.float32))
    negf = jnp.sum(neg.astype(jnp.float32))
    stats_ref[...] += jnp.stack([posf, negf, loc_part, posce_part]).reshape(1, 1, 4)


def _phase2(ce_ref, stats_ref, out_ref):
    v = ce_ref[...]
    vi = lax.bitcast_convert_type(v, jnp.int32)
    stats = stats_ref[...]
    num_pos = stats[:, 0:1]
    num_neg = stats[:, 1:2]
    k = jnp.minimum(NEG_POS_RATIO * num_pos, num_neg)

    B = v.shape[0]
    lo = jnp.zeros((B, 1), jnp.int32)
    hi = jnp.full((B, 1), 2**31 - 2, jnp.int32)

    def body(_, carry):
        lo, hi = carry
        mid = lo + (hi - lo + 1) // 2
        cnt = jnp.sum((vi >= mid).astype(jnp.float32), axis=1, keepdims=True)
        ok = cnt >= k
        return jnp.where(ok, mid, lo), jnp.where(ok, hi, mid - 1)

    lo, hi = lax.fori_loop(0, 31, body, (lo, hi))
    tstar = lo
    gt = vi > tstar
    c_gt = jnp.sum(gt.astype(jnp.float32), axis=1, keepdims=True)
    sum_gt = jnp.sum(jnp.where(gt, v, 0.0), axis=1, keepdims=True)
    tf = lax.bitcast_convert_type(tstar, jnp.float32)
    topk = sum_gt + (k - c_gt) * tf
    topk = jnp.where(k > 0.5, topk, 0.0)

    loc_loss = jnp.sum(stats[:, 2])
    posce = jnp.sum(stats[:, 3])
    n = jnp.maximum(jnp.sum(num_pos), 1.0)
    out_ref[...] = jnp.reshape((loc_loss + posce + jnp.sum(topk)) / n, (1, 1))


def kernel(loc_preds, cls_preds, loc_targets, cls_targets):
    B, P = loc_preds.shape[0], loc_preds.shape[1]
    C = 12288
    NC = (P + C - 1) // C
    Ppad = NC * C

    cls_t = jnp.pad(cls_targets.astype(jnp.int32), ((0, 0), (0, Ppad - P)),
                    constant_values=-1).reshape(B * NC, 1, C)

    loc_p_t = jnp.transpose(loc_preds, (0, 2, 1))
    loc_t_t = jnp.transpose(loc_targets, (0, 2, 1))
    cls_p_t = jnp.transpose(cls_preds, (0, 2, 1))

    ce_neg, stats = pl.pallas_call(
        functools.partial(_phase1, C=C),
        grid=(B, NC),
        in_specs=[
            pl.BlockSpec((1, 4, C), lambda b, nc: (b, 0, nc)),
            pl.BlockSpec((1, 4, C), lambda b, nc: (b, 0, nc)),
            pl.BlockSpec((1, NUM_CLASSES + 1, C), lambda b, nc: (b, 0, nc)),
            pl.BlockSpec((1, 1, C), lambda b, nc: (b * NC + nc, 0, 0)),
        ],
        out_specs=[
            pl.BlockSpec((1, 1, C), lambda b, nc: (b * NC + nc, 0, 0)),
            pl.BlockSpec((1, 1, 4), lambda b, nc: (b, 0, 0)),
        ],
        out_shape=[
            jax.ShapeDtypeStruct((B * NC, 1, C), jnp.float32),
            jax.ShapeDtypeStruct((B, 1, 4), jnp.float32),
        ],
    )(loc_p_t, loc_t_t, cls_p_t, cls_t)

    out = pl.pallas_call(
        _phase2,
        out_shape=jax.ShapeDtypeStruct((1, 1), jnp.float32),
    )(ce_neg.reshape(B, Ppad), stats.reshape(B, 4))

    return out.reshape(())


# speedup vs baseline: 5.5313x; 1.1076x over previous
import functools

import jax
import jax.numpy as jnp
from jax import lax
from jax.experimental import pallas as pl

NEG_POS_RATIO = 3
NUM_CLASSES = 20


def _phase1(loc_p_ref, loc_t_ref, cls_p_ref, cls_t_ref, ce_ref, stats_ref, *, C):
    nc = pl.program_id(1)

    t = cls_t_ref[0]
    pos = t > 0
    neg = t == 0

    d = loc_p_ref[0] - loc_t_ref[0]
    ad = jnp.abs(d)
    sl1 = jnp.where(ad < 1.0, 0.5 * d * d, ad - 0.5)
    sl = jnp.sum(sl1, axis=0, keepdims=True)
    loc_part = jnp.sum(jnp.where(pos, sl, 0.0))

    x = cls_p_ref[0]
    m = jnp.max(x, axis=0, keepdims=True)
    lse = m + jnp.log(jnp.sum(jnp.exp(x - m), axis=0, keepdims=True))
    ci = lax.broadcasted_iota(jnp.int32, x.shape, 0)
    xt = jnp.sum(jnp.where(ci == t, x, 0.0), axis=0, keepdims=True)
    ce = lse - xt

    posce_part = jnp.sum(jnp.where(pos, ce, 0.0))
    ce_ref[0] = jnp.where(neg, ce, -jnp.inf)

    @pl.when(nc == 0)
    def _():
        stats_ref[...] = jnp.zeros_like(stats_ref)

    posf = jnp.sum(pos.astype(jnp.float32))
    negf = jnp.sum(neg.astype(jnp.float32))
    stats_ref[...] += jnp.stack([posf, negf, loc_part, posce_part]).reshape(1, 1, 4)


def _phase2(ce_ref, stats_ref, out_ref):
    v = ce_ref[...]
    vi = lax.bitcast_convert_type(v, jnp.int32)
    stats = stats_ref[...]
    num_pos = stats[:, 0:1]
    num_neg = stats[:, 1:2]
    k = jnp.minimum(NEG_POS_RATIO * num_pos, num_neg)

    B = v.shape[0]
    lo = jnp.zeros((B, 1), jnp.int32)
    hi = jnp.full((B, 1), 2**31 - 2, jnp.int32)

    def body(_, carry):
        lo, hi = carry
        mid = lo + (hi - lo + 1) // 2
        cnt = jnp.sum((vi >= mid).astype(jnp.float32), axis=1, keepdims=True)
        ok = cnt >= k
        return jnp.where(ok, mid, lo), jnp.where(ok, hi, mid - 1)

    lo, hi = lax.fori_loop(0, 31, body, (lo, hi))
    tstar = lo
    gt = vi > tstar
    c_gt = jnp.sum(gt.astype(jnp.float32), axis=1, keepdims=True)
    sum_gt = jnp.sum(jnp.where(gt, v, 0.0), axis=1, keepdims=True)
    tf = lax.bitcast_convert_type(tstar, jnp.float32)
    topk = sum_gt + (k - c_gt) * tf
    topk = jnp.where(k > 0.5, topk, 0.0)

    loc_loss = jnp.sum(stats[:, 2])
    posce = jnp.sum(stats[:, 3])
    n = jnp.maximum(jnp.sum(num_pos), 1.0)
    out_ref[...] = jnp.reshape((loc_loss + posce + jnp.sum(topk)) / n, (1, 1))


def kernel(loc_preds, cls_preds, loc_targets, cls_targets):
    B, P = loc_preds.shape[0], loc_preds.shape[1]
    C = 24576
    NC = (P + C - 1) // C
    Ppad = NC * C

    cls_t = jnp.pad(cls_targets.astype(jnp.int32), ((0, 0), (0, Ppad - P)),
                    constant_values=-1).reshape(B * NC, 1, C)

    loc_p_t = jnp.transpose(loc_preds, (0, 2, 1))
    loc_t_t = jnp.transpose(loc_targets, (0, 2, 1))
    cls_p_t = jnp.transpose(cls_preds, (0, 2, 1))

    ce_neg, stats = pl.pallas_call(
        functools.partial(_phase1, C=C),
        grid=(B, NC),
        in_specs=[
            pl.BlockSpec((1, 4, C), lambda b, nc: (b, 0, nc)),
            pl.BlockSpec((1, 4, C), lambda b, nc: (b, 0, nc)),
            pl.BlockSpec((1, NUM_CLASSES + 1, C), lambda b, nc: (b, 0, nc)),
            pl.BlockSpec((1, 1, C), lambda b, nc: (b * NC + nc, 0, 0)),
        ],
        out_specs=[
            pl.BlockSpec((1, 1, C), lambda b, nc: (b * NC + nc, 0, 0)),
            pl.BlockSpec((1, 1, 4), lambda b, nc: (b, 0, 0)),
        ],
        out_shape=[
            jax.ShapeDtypeStruct((B * NC, 1, C), jnp.float32),
            jax.ShapeDtypeStruct((B, 1, 4), jnp.float32),
        ],
    )(loc_p_t, loc_t_t, cls_p_t, cls_t)

    out = pl.pallas_call(
        _phase2,
        out_shape=jax.ShapeDtypeStruct((1, 1), jnp.float32),
    )(ce_neg.reshape(B, Ppad), stats.reshape(B, 4))

    return out.reshape(())


# speedup vs baseline: 5.7181x; 1.0338x over previous
import functools

import jax
import jax.numpy as jnp
from jax import lax
from jax.experimental import pallas as pl
from jax.experimental.pallas import tpu as pltpu

NEG_POS_RATIO = 3
NUM_CLASSES = 20


def _fused(loc_p_ref, loc_t_ref, cls_p_ref, cls_t_ref, out_ref, ce_s_ref,
           *, C, B):
    b = pl.program_id(0)

    t = cls_t_ref[0]
    pos = t > 0
    neg = t == 0

    d = loc_p_ref[0] - loc_t_ref[0]
    ad = jnp.abs(d)
    sl1 = jnp.where(ad < 1.0, 0.5 * d * d, ad - 0.5)
    sl = jnp.sum(sl1, axis=0, keepdims=True)
    loc_part = jnp.sum(jnp.where(pos, sl, 0.0))

    x = cls_p_ref[0]
    m = jnp.max(x, axis=0, keepdims=True)
    lse = m + jnp.log(jnp.sum(jnp.exp(x - m), axis=0, keepdims=True))
    ci = lax.broadcasted_iota(jnp.int32, x.shape, 0)
    xt = jnp.sum(jnp.where(ci == t, x, 0.0), axis=0, keepdims=True)
    ce = lse - xt

    posce_part = jnp.sum(jnp.where(pos, ce, 0.0))
    posf = jnp.sum(pos.astype(jnp.float32))
    negf = jnp.sum(neg.astype(jnp.float32))

    enc = jnp.where(neg, ce, -jnp.inf)
    cidx = lax.broadcasted_iota(jnp.int32, (1, C), 1)
    for i, val in enumerate((posf, negf, loc_part, posce_part)):
        enc = jnp.where(cidx == C - 4 + i, -1.0 - val, enc)
    ce_s_ref[pl.ds(b, 1), :] = enc

    @pl.when(b == B - 1)
    def _():
        v = ce_s_ref[...]
        vi = lax.bitcast_convert_type(v, jnp.int32)
        st = -1.0 - v[:, C - 4:]
        num_pos = st[:, 0:1]
        num_neg = st[:, 1:2]
        k = jnp.minimum(NEG_POS_RATIO * num_pos, num_neg)

        lo = jnp.zeros((B, 1), jnp.int32)
        hi = jnp.full((B, 1), 2**31 - 2, jnp.int32)

        def body(_, carry):
            lo, hi = carry
            mid = lo + (hi - lo + 1) // 2
            cnt = jnp.sum((vi >= mid).astype(jnp.float32), axis=1, keepdims=True)
            ok = cnt >= k
            return jnp.where(ok, mid, lo), jnp.where(ok, hi, mid - 1)

        lo, hi = lax.fori_loop(0, 31, body, (lo, hi))
        tstar = lo
        gt = vi > tstar
        c_gt = jnp.sum(gt.astype(jnp.float32), axis=1, keepdims=True)
        sum_gt = jnp.sum(jnp.where(gt, v, 0.0), axis=1, keepdims=True)
        tf = lax.bitcast_convert_type(tstar, jnp.float32)
        topk = sum_gt + (k - c_gt) * tf
        topk = jnp.where(k > 0.5, topk, 0.0)

        loc_loss = jnp.sum(st[:, 2])
        posce = jnp.sum(st[:, 3])
        n = jnp.maximum(jnp.sum(num_pos), 1.0)
        out_ref[...] = jnp.reshape((loc_loss + posce + jnp.sum(topk)) / n, (1, 1))


def kernel(loc_preds, cls_preds, loc_targets, cls_targets):
    B, P = loc_preds.shape[0], loc_preds.shape[1]
    C = 24576

    cls_t = jnp.pad(cls_targets.astype(jnp.int32), ((0, 0), (0, C - P)),
                    constant_values=-1).reshape(B, 1, C)

    loc_p_t = jnp.transpose(loc_preds, (0, 2, 1))
    loc_t_t = jnp.transpose(loc_targets, (0, 2, 1))
    cls_p_t = jnp.transpose(cls_preds, (0, 2, 1))

    out = pl.pallas_call(
        functools.partial(_fused, C=C, B=B),
        grid=(B,),
        in_specs=[
            pl.BlockSpec((1, 4, C), lambda b: (b, 0, 0)),
            pl.BlockSpec((1, 4, C), lambda b: (b, 0, 0)),
            pl.BlockSpec((1, NUM_CLASSES + 1, C), lambda b: (b, 0, 0)),
            pl.BlockSpec((1, 1, C), lambda b: (b, 0, 0)),
        ],
        out_specs=pl.BlockSpec((1, 1), lambda b: (0, 0)),
        out_shape=jax.ShapeDtypeStruct((1, 1), jnp.float32),
        scratch_shapes=[pltpu.VMEM((B, C), jnp.float32)],
    )(loc_p_t, loc_t_t, cls_p_t, cls_t)

    return out.reshape(())
